# baseline (speedup 1.0000x reference)
import jax
import jax.numpy as jnp
from jax.experimental import pallas as pl


_BLOCK_ROWS = 2000


def _dot(a, b):
    return jnp.dot(a.astype(jnp.bfloat16), b.astype(jnp.bfloat16),
                   preferred_element_type=jnp.float32)


def _fused_mlp_kernel(x_ref, w_in_ref, w_h1_ref, w_h2_ref, w_out_ref,
                      w_cls_ref, b_cls_ref, out_ref):
    x = x_ref[...]
    h = jax.nn.relu(_dot(x, w_in_ref[...]))
    h = jax.nn.relu(_dot(h, w_h1_ref[...]))
    h = jax.nn.relu(_dot(h, w_h2_ref[...]))
    h = _dot(h, w_out_ref[...])
    y = _dot(h, w_cls_ref[...])
    out_ref[...] = y + b_cls_ref[...]


def kernel(h_0, W_in, W_h1, W_h2, W_out, W_cls, b_cls):
    B, N, D_in = h_0.shape
    D_h = W_in.shape[1]
    D_out = W_cls.shape[1]
    rows = B * N
    x = h_0.reshape(rows, D_in)
    b2 = b_cls.reshape(1, D_out)

    block_rows = _BLOCK_ROWS if rows % _BLOCK_ROWS == 0 else rows
    grid = (rows // block_rows,)

    def w_spec(shape):
        return pl.BlockSpec(shape, lambda i: (0, 0))

    y = pl.pallas_call(
        _fused_mlp_kernel,
        grid=grid,
        in_specs=[
            pl.BlockSpec((block_rows, D_in), lambda i: (i, 0)),
            w_spec((D_in, D_h)),
            w_spec((D_h, D_h)),
            w_spec((D_h, D_h)),
            w_spec((D_h, D_h)),
            w_spec((D_h, D_out)),
            w_spec((1, D_out)),
        ],
        out_specs=pl.BlockSpec((block_rows, D_out), lambda i: (i, 0)),
        out_shape=jax.ShapeDtypeStruct((rows, D_out), jnp.float32),
    )(x, W_in, W_h1, W_h2, W_out, W_cls, b2)

    return y.reshape(B, N, D_out)


# speedup vs baseline: 1.0947x; 1.0947x over previous
import jax
import jax.numpy as jnp
from jax.experimental import pallas as pl
from jax.experimental.pallas import tpu as pltpu


_BLOCK_ROWS = 5000


def _dot(a, b):
    return jnp.dot(a.astype(jnp.bfloat16), b.astype(jnp.bfloat16),
                   preferred_element_type=jnp.float32)


def _fused_mlp_kernel(x_ref, w_in_ref, w_h1_ref, w_h2_ref, w_out_ref,
                      w_cls_ref, b_cls_ref, out_ref):
    x = x_ref[...]
    h = jax.nn.relu(_dot(x, w_in_ref[...]))
    h = jax.nn.relu(_dot(h, w_h1_ref[...]))
    h = jax.nn.relu(_dot(h, w_h2_ref[...]))
    h = _dot(h, w_out_ref[...])
    y = _dot(h, w_cls_ref[...])
    out_ref[...] = y + b_cls_ref[...]


def kernel(h_0, W_in, W_h1, W_h2, W_out, W_cls, b_cls):
    B, N, D_in = h_0.shape
    D_h = W_in.shape[1]
    D_out = W_cls.shape[1]
    rows = B * N
    x = h_0.reshape(rows, D_in)
    b2 = b_cls.reshape(1, D_out)

    block_rows = _BLOCK_ROWS if rows % _BLOCK_ROWS == 0 else rows
    grid = (rows // block_rows,)

    def w_spec(shape):
        return pl.BlockSpec(shape, lambda i: (0, 0))

    y = pl.pallas_call(
        _fused_mlp_kernel,
        grid=grid,
        in_specs=[
            pl.BlockSpec((block_rows, D_in), lambda i: (i, 0)),
            w_spec((D_in, D_h)),
            w_spec((D_h, D_h)),
            w_spec((D_h, D_h)),
            w_spec((D_h, D_h)),
            w_spec((D_h, D_out)),
            w_spec((1, D_out)),
        ],
        out_specs=pl.BlockSpec((block_rows, D_out), lambda i: (i, 0)),
        out_shape=jax.ShapeDtypeStruct((rows, D_out), jnp.float32),
        compiler_params=pltpu.CompilerParams(
            dimension_semantics=("parallel",)),
    )(x, W_in, W_h1, W_h2, W_out, W_cls, b2)

    return y.reshape(B, N, D_out)


# speedup vs baseline: 1.3240x; 1.2095x over previous
import jax
import jax.numpy as jnp
from jax.experimental import pallas as pl
from jax.experimental.pallas import tpu as pltpu


_BLOCK_N = 5000


def _dot(a, b):
    return jnp.dot(a.astype(jnp.bfloat16), b.astype(jnp.bfloat16),
                   preferred_element_type=jnp.float32)


def _fused_mlp_kernel(x_ref, w_in_ref, w_h1_ref, w_h2_ref, w_out_ref,
                      w_cls_ref, b_cls_ref, out_ref):
    x = x_ref[0]
    h = jax.nn.relu(_dot(x, w_in_ref[...]))
    h = jax.nn.relu(_dot(h, w_h1_ref[...]))
    h = jax.nn.relu(_dot(h, w_h2_ref[...]))
    h = _dot(h, w_out_ref[...])
    y = _dot(h, w_cls_ref[...])
    out_ref[0] = y + b_cls_ref[...]


def kernel(h_0, W_in, W_h1, W_h2, W_out, W_cls, b_cls):
    B, N, D_in = h_0.shape
    D_h = W_in.shape[1]
    D_out = W_cls.shape[1]
    b2 = b_cls.reshape(1, D_out)

    block_n = _BLOCK_N if N % _BLOCK_N == 0 else N
    grid = (B, N // block_n)

    def w_spec(shape):
        return pl.BlockSpec(shape, lambda b, i: (0, 0))

    return pl.pallas_call(
        _fused_mlp_kernel,
        grid=grid,
        in_specs=[
            pl.BlockSpec((1, block_n, D_in), lambda b, i: (b, i, 0)),
            w_spec((D_in, D_h)),
            w_spec((D_h, D_h)),
            w_spec((D_h, D_h)),
            w_spec((D_h, D_h)),
            w_spec((D_h, D_out)),
            w_spec((1, D_out)),
        ],
        out_specs=pl.BlockSpec((1, block_n, D_out), lambda b, i: (b, i, 0)),
        out_shape=jax.ShapeDtypeStruct((B, N, D_out), jnp.float32),
        compiler_params=pltpu.CompilerParams(
            dimension_semantics=("parallel", "parallel")),
    )(h_0, W_in, W_h1, W_h2, W_out, W_cls, b2)


# speedup vs baseline: 1.9847x; 1.4991x over previous
import jax
import jax.numpy as jnp
from jax.experimental import pallas as pl
from jax.experimental.pallas import tpu as pltpu


_BLOCK_N = 5000


def _dot(a, b):
    return jnp.dot(a.astype(jnp.bfloat16), b.astype(jnp.bfloat16),
                   preferred_element_type=jnp.float32)


def _fused_mlp_kernel(x_ref, w_in_ref, w_h1_ref, w_h2_ref, w_out_ref,
                      w_cls_t_ref, b_cls_ref, out_ref):
    x = x_ref[0]
    h = jax.nn.relu(_dot(x, w_in_ref[...]))
    h = jax.nn.relu(_dot(h, w_h1_ref[...]))
    h = jax.nn.relu(_dot(h, w_h2_ref[...]))
    h = _dot(h, w_out_ref[...])
    y_t = jax.lax.dot_general(
        w_cls_t_ref[...].astype(jnp.bfloat16), h.astype(jnp.bfloat16),
        dimension_numbers=(((1,), (1,)), ((), ())),
        preferred_element_type=jnp.float32)
    b = jax.lax.broadcast_in_dim(b_cls_ref[0], y_t.shape, (0,))
    out_ref[0] = y_t + b


def kernel(h_0, W_in, W_h1, W_h2, W_out, W_cls, b_cls):
    B, N, D_in = h_0.shape
    D_h = W_in.shape[1]
    D_out = W_cls.shape[1]
    W_cls_t = W_cls.T
    b2 = b_cls.reshape(1, D_out)

    grid = (B,)

    def w_spec(shape):
        return pl.BlockSpec(shape, lambda b: (0, 0))

    y_t = pl.pallas_call(
        _fused_mlp_kernel,
        grid=grid,
        in_specs=[
            pl.BlockSpec((1, N, D_in), lambda b: (b, 0, 0)),
            w_spec((D_in, D_h)),
            w_spec((D_h, D_h)),
            w_spec((D_h, D_h)),
            w_spec((D_h, D_h)),
            w_spec((D_out, D_h)),
            w_spec((1, D_out)),
        ],
        out_specs=pl.BlockSpec((1, D_out, N), lambda b: (b, 0, 0)),
        out_shape=jax.ShapeDtypeStruct((B, D_out, N), jnp.float32),
        compiler_params=pltpu.CompilerParams(
            dimension_semantics=("parallel",)),
    )(h_0, W_in, W_h1, W_h2, W_out, W_cls_t, b2)

    return jnp.transpose(y_t, (0, 2, 1))
